# baseline (speedup 1.0000x reference)
import functools
import math

import jax
import jax.numpy as jnp
from jax import lax
from jax.experimental import pallas as pl
from jax.experimental.pallas import tpu as pltpu
from jax.experimental.pallas import tpu_sc as plsc

NH = 8
HD = 24
SCALE = 1.0 / math.sqrt(HD)
NEG = -1e30

_pallas_call = pl.pallas_call


def _tok(x):
    _, c, h, w = x.shape
    t = x.reshape(c, h * w).T.reshape(h * w, NH, HD)
    return t.transpose(1, 0, 2)


def _cells(x):
    _, c, h, w = x.shape
    t = x.reshape(c, h // 2, 2, w // 2, 2)
    t = t.transpose(2, 4, 1, 3, 0)
    t = t.reshape(4 * (h // 2) * (w // 2), NH, HD)
    return t.transpose(1, 0, 2)


def _dot3(a, b):
    ah = a.astype(jnp.bfloat16)
    al = (a - ah.astype(jnp.float32)).astype(jnp.bfloat16)
    bh = b.astype(jnp.bfloat16)
    bl = (b - bh.astype(jnp.float32)).astype(jnp.bfloat16)
    f32 = jnp.float32
    return (jnp.dot(ah, bh, preferred_element_type=f32)
            + jnp.dot(ah, bl, preferred_element_type=f32)
            + jnp.dot(al, bh, preferred_element_type=f32))


def _dot2r(a, b):
    ah = a.astype(jnp.bfloat16)
    bh = b.astype(jnp.bfloat16)
    bl = (b - bh.astype(jnp.float32)).astype(jnp.bfloat16)
    f32 = jnp.float32
    return (jnp.dot(ah, bh, preferred_element_type=f32)
            + jnp.dot(ah, bl, preferred_element_type=f32))

def _topk_iter(logits, k, ncols):
    rows = logits.shape[0]
    cols = jax.lax.broadcasted_iota(jnp.int32, (rows, ncols), 1)
    cur = logits
    idxs = []
    for _ in range(k):
        m = jnp.max(cur, axis=1, keepdims=True)
        idx = jnp.min(jnp.where(cur >= m, cols, ncols), axis=1)
        idxs.append(idx)
        cur = jnp.where(cols == idx[:, None], NEG, cur)
    out = jnp.zeros((rows, k), jnp.int32)
    sel = jax.lax.broadcasted_iota(jnp.int32, (rows, k), 1)
    for t in range(k):
        out = jnp.where(sel == t, idxs[t][:, None], out)
    return out


def _stage_a(q_ref, k_ref, v_ref, m_ref, i_ref):
    q = q_ref[0]
    k = k_ref[0]
    v = v_ref[0]
    s = jnp.dot(q.astype(jnp.bfloat16), k.astype(jnp.bfloat16).T, preferred_element_type=jnp.float32) * SCALE
    mx = jnp.max(s, axis=1, keepdims=True)
    e = jnp.exp(s - mx)
    p = e / jnp.sum(e, axis=1, keepdims=True)
    m_ref[0] = jnp.dot(p.astype(jnp.bfloat16), v.astype(jnp.bfloat16), preferred_element_type=jnp.float32)
    i_ref[0] = _topk_iter(s, 16, 256)


def _stage_b(q_ref, k_ref, v_ref, i2_ref, m_ref, i1_ref):
    qh = q_ref[0]
    kh = k_ref[0]
    vh = v_ref[0]
    i2 = i2_ref[0]
    dn = (((0,), (0,)), ((), ()))
    s = jax.lax.dot_general(qh, kh, dn, preferred_element_type=jnp.float32,
                            precision=jax.lax.Precision.HIGHEST) * SCALE
    col = jax.lax.broadcasted_iota(jnp.int32, (1024, 1024), 1)
    cellcol = ((col >> 5) >> 1) * 16 + ((col & 31) >> 1)
    mask = jnp.zeros((1024, 1024), jnp.bool_)
    for t in range(16):
        mask = jnp.logical_or(mask, i2[:, t][:, None] == cellcol)
    L = jnp.where(mask, s, NEG)
    mx = jnp.max(L, axis=1, keepdims=True)
    e = jnp.exp(L - mx)
    dnc = (((1,), (1,)), ((), ()))
    eh = e.astype(jnp.bfloat16)
    vbh = vh.astype(jnp.bfloat16)
    vbl = (vh - vbh.astype(jnp.float32)).astype(jnp.bfloat16)
    num = (jax.lax.dot_general(eh, vbh, dnc, preferred_element_type=jnp.float32)
           + jax.lax.dot_general(eh, vbl, dnc, preferred_element_type=jnp.float32))
    m_ref[0] = num / jnp.sum(e, axis=1, keepdims=True)
    i1_ref[0] = _topk_iter(L, 8, 1024)


def _sc_gather(table, idx):
    B = idx.shape[0]
    D = table.shape[1]
    NW = 32
    CH = 256
    nch = B // (NW * CH)
    mesh = plsc.VectorSubcoreMesh(core_axis_name="c", subcore_axis_name="s")

    @functools.partial(
        pl.kernel, mesh=mesh,
        out_type=jax.ShapeDtypeStruct((B, D), jnp.float32),
        scratch_types=[
            pltpu.VMEM((CH,), jnp.int32),
            pltpu.VMEM((CH, D), jnp.float32),
            pltpu.SemaphoreType.DMA,
        ],
    )
    def gk(table_hbm, idx_hbm, out_hbm, idx_v, rows_v, sem):
        wid = lax.axis_index("s") * 2 + lax.axis_index("c")
        base = wid * (B // NW)
        for c in range(nch):
            off = base + c * CH
            pltpu.sync_copy(idx_hbm.at[pl.ds(off, CH)], idx_v)
            pltpu.async_copy(table_hbm.at[idx_v], rows_v, sem).wait()
            pltpu.sync_copy(rows_v, out_hbm.at[pl.ds(off, CH)])

    return gk(table, idx)


def _stage_c(q_ref, k_ref, v_ref, base_ref, w2_ref, o_ref, *, P):
    q = q_ref[0]
    kk = k_ref[0]
    vv = v_ref[0]
    qb = jnp.broadcast_to(q[:, :, None, :], (P, 4, 32, HD))
    kb = jnp.broadcast_to(kk[:, None, :, :], (P, 4, 32, HD))
    s = jnp.sum(qb * kb, axis=-1) * SCALE
    mx = jnp.max(s, axis=-1, keepdims=True)
    e = jnp.exp(s - mx)
    den = jnp.sum(e, axis=-1, keepdims=True)
    pb = jnp.broadcast_to(e[..., None], (P, 4, 32, HD))
    vb = jnp.broadcast_to(vv[:, None, :, :], (P, 4, 32, HD))
    m0 = jnp.sum(pb * vb, axis=2) / den
    out = base_ref[0][:, None, :] + w2_ref[0, 0] * m0
    o_ref[0] = out


def kernel(q0, q1, q2, k0, k1, k2, v0, v1, v2, weight):
    f32 = jnp.float32
    q2t, k2t, v2t = _tok(q2), _tok(k2), _tok(v2)
    q0c, k0c, v0c = _cells(q0), _cells(k0), _cells(v0)

    m2, i2 = _pallas_call(
        _stage_a,
        grid=(NH,),
        in_specs=[pl.BlockSpec((1, 256, HD), lambda h: (h, 0, 0))] * 3,
        out_specs=[pl.BlockSpec((1, 256, HD), lambda h: (h, 0, 0)),
                   pl.BlockSpec((1, 256, 16), lambda h: (h, 0, 0))],
        out_shape=[jax.ShapeDtypeStruct((NH, 256, HD), f32),
                   jax.ShapeDtypeStruct((NH, 256, 16), jnp.int32)],
    )(q2t, k2t, v2t)

    q1r = q1.reshape(NH, HD, 1024)
    k1r = k1.reshape(NH, HD, 1024)
    v1r = v1.reshape(NH, HD, 1024)
    i2tok = jnp.broadcast_to(
        i2.reshape(NH, 16, 1, 16, 1, 16), (NH, 16, 2, 16, 2, 16)
    ).reshape(NH, 1024, 16)
    m1, i1 = _pallas_call(
        _stage_b,
        grid=(NH,),
        in_specs=[pl.BlockSpec((1, HD, 1024), lambda h: (h, 0, 0))] * 3
        + [pl.BlockSpec((1, 1024, 16), lambda h: (h, 0, 0))],
        out_specs=[pl.BlockSpec((1, 1024, HD), lambda h: (h, 0, 0)),
                   pl.BlockSpec((1, 1024, 8), lambda h: (h, 0, 0))],
        out_shape=[jax.ShapeDtypeStruct((NH, 1024, HD), f32),
                   jax.ShapeDtypeStruct((NH, 1024, 8), jnp.int32)],
    )(q1r, k1r, v1r, i2tok)

    wsm = jax.nn.softmax(weight, axis=0)
    m2up = jnp.broadcast_to(
        m2.reshape(NH, 16, 1, 16, 1, HD), (NH, 16, 2, 16, 2, HD)
    ).reshape(NH, 1024, HD)
    base = wsm[0] * m2up + wsm[1] * m1
    i1tok = i1

    kv0 = jnp.concatenate([
        k0c.reshape(NH, 4, 1024, HD).transpose(0, 2, 1, 3).reshape(NH * 1024, 4 * HD),
        v0c.reshape(NH, 4, 1024, HD).transpose(0, 2, 1, 3).reshape(NH * 1024, 4 * HD),
        jnp.zeros((NH * 1024, 64), jnp.float32),
    ], axis=-1)
    hoff = (jnp.arange(NH, dtype=jnp.int32) * 1024)[:, None, None]
    idx = (i1tok + hoff).reshape(NH * 1024 * 8)
    g = _sc_gather(kv0, idx).reshape(NH, 1024, 8, 256)
    kg = g[..., 0:96].reshape(NH, 1024, 32, HD)
    vg = g[..., 96:192].reshape(NH, 1024, 32, HD)

    P = 256
    npb = 1024 // P
    q0p = q0c.reshape(NH, 4, 1024, HD).transpose(0, 2, 1, 3)
    w2 = wsm[2].reshape(1, 1)
    outf = _pallas_call(
        functools.partial(_stage_c, P=P),
        grid=(NH, npb),
        in_specs=[
            pl.BlockSpec((1, P, 4, HD), lambda h, pb: (h, pb, 0, 0)),
            pl.BlockSpec((1, P, 32, HD), lambda h, pb: (h, pb, 0, 0)),
            pl.BlockSpec((1, P, 32, HD), lambda h, pb: (h, pb, 0, 0)),
            pl.BlockSpec((1, P, HD), lambda h, pb: (h, pb, 0)),
            pl.BlockSpec((1, 1), lambda h, pb: (0, 0)),
        ],
        out_specs=pl.BlockSpec((1, P, 4, HD), lambda h, pb: (h, pb, 0, 0)),
        out_shape=jax.ShapeDtypeStruct((NH, 1024, 4, HD), f32),
    )(q0p, kg, vg, base, w2)

    out = outf.reshape(NH, 32, 32, 2, 2, HD).transpose(0, 1, 3, 2, 4, 5)
    out = out.reshape(NH, 4096, HD).transpose(1, 0, 2)[None]
    return out


# speedup vs baseline: 1.1129x; 1.1129x over previous
import functools
import math

import jax
import jax.numpy as jnp
from jax import lax
from jax.experimental import pallas as pl
from jax.experimental.pallas import tpu as pltpu
from jax.experimental.pallas import tpu_sc as plsc

NH = 8
HD = 24
SCALE = 1.0 / math.sqrt(HD)
NEG = -1e30

_pallas_call = pl.pallas_call


def _tok(x):
    _, c, h, w = x.shape
    t = x.reshape(c, h * w).T.reshape(h * w, NH, HD)
    return t.transpose(1, 0, 2)


def _cells(x):
    _, c, h, w = x.shape
    t = x.reshape(c, h // 2, 2, w // 2, 2)
    t = t.transpose(2, 4, 1, 3, 0)
    t = t.reshape(4 * (h // 2) * (w // 2), NH, HD)
    return t.transpose(1, 0, 2)


def _dot3(a, b):
    ah = a.astype(jnp.bfloat16)
    al = (a - ah.astype(jnp.float32)).astype(jnp.bfloat16)
    bh = b.astype(jnp.bfloat16)
    bl = (b - bh.astype(jnp.float32)).astype(jnp.bfloat16)
    f32 = jnp.float32
    return (jnp.dot(ah, bh, preferred_element_type=f32)
            + jnp.dot(ah, bl, preferred_element_type=f32)
            + jnp.dot(al, bh, preferred_element_type=f32))


def _dot2r(a, b):
    ah = a.astype(jnp.bfloat16)
    bh = b.astype(jnp.bfloat16)
    bl = (b - bh.astype(jnp.float32)).astype(jnp.bfloat16)
    f32 = jnp.float32
    return (jnp.dot(ah, bh, preferred_element_type=f32)
            + jnp.dot(ah, bl, preferred_element_type=f32))

def _topk_iter(logits, k, ncols):
    rows = logits.shape[0]
    cols = jax.lax.broadcasted_iota(jnp.int32, (rows, ncols), 1)
    cur = logits
    idxs = []
    for _ in range(k):
        m = jnp.max(cur, axis=1, keepdims=True)
        idx = jnp.min(jnp.where(cur >= m, cols, ncols), axis=1)
        idxs.append(idx)
        cur = jnp.where(cols == idx[:, None], NEG, cur)
    out = jnp.zeros((rows, k), jnp.int32)
    sel = jax.lax.broadcasted_iota(jnp.int32, (rows, k), 1)
    for t in range(k):
        out = jnp.where(sel == t, idxs[t][:, None], out)
    return out


def _stage_a(q_ref, k_ref, v_ref, m_ref, i_ref):
    q = q_ref[0]
    k = k_ref[0]
    v = v_ref[0]
    s = jnp.dot(q.astype(jnp.bfloat16), k.astype(jnp.bfloat16).T, preferred_element_type=jnp.float32) * SCALE
    mx = jnp.max(s, axis=1, keepdims=True)
    e = jnp.exp(s - mx)
    p = e / jnp.sum(e, axis=1, keepdims=True)
    m_ref[0] = jnp.dot(p.astype(jnp.bfloat16), v.astype(jnp.bfloat16), preferred_element_type=jnp.float32)
    i_ref[0] = _topk_iter(s, 16, 256)


def _stage_b(q_ref, k_ref, v_ref, i2_ref, m_ref, i1_ref):
    qh = q_ref[0]
    kh = k_ref[0]
    vh = v_ref[0]
    i2 = i2_ref[0]
    dn = (((0,), (0,)), ((), ()))
    s = jax.lax.dot_general(qh, kh, dn, preferred_element_type=jnp.float32,
                            precision=jax.lax.Precision.HIGHEST) * SCALE
    col = jax.lax.broadcasted_iota(jnp.int32, (1024, 1024), 1)
    cellcol = ((col >> 5) >> 1) * 16 + ((col & 31) >> 1)
    mask = jnp.zeros((1024, 1024), jnp.bool_)
    for t in range(16):
        mask = jnp.logical_or(mask, i2[:, t][:, None] == cellcol)
    L = jnp.where(mask, s, NEG)
    mx = jnp.max(L, axis=1, keepdims=True)
    e = jnp.exp(L - mx)
    dnc = (((1,), (1,)), ((), ()))
    eh = e.astype(jnp.bfloat16)
    vbh = vh.astype(jnp.bfloat16)
    vbl = (vh - vbh.astype(jnp.float32)).astype(jnp.bfloat16)
    num = (jax.lax.dot_general(eh, vbh, dnc, preferred_element_type=jnp.float32)
           + jax.lax.dot_general(eh, vbl, dnc, preferred_element_type=jnp.float32))
    m_ref[0] = num / jnp.sum(e, axis=1, keepdims=True)
    i1_ref[0] = _topk_iter(L, 8, 1024)


def _sc_gather2(ktab, vtab, idx):
    B = idx.shape[0]
    D = ktab.shape[1]
    NW = 32
    CH = 256
    nch = B // (NW * CH)
    mesh = plsc.VectorSubcoreMesh(core_axis_name="c", subcore_axis_name="s")

    @functools.partial(
        pl.kernel, mesh=mesh,
        out_type=[jax.ShapeDtypeStruct((B, D), jnp.float32),
                  jax.ShapeDtypeStruct((B, D), jnp.float32)],
        scratch_types=[
            pltpu.VMEM((CH,), jnp.int32),
            pltpu.VMEM((CH, D), jnp.float32),
            pltpu.VMEM((CH, D), jnp.float32),
            pltpu.SemaphoreType.DMA,
            pltpu.SemaphoreType.DMA,
        ],
    )
    def gk(ktab_hbm, vtab_hbm, idx_hbm, ko_hbm, vo_hbm,
           idx_v, krows, vrows, sem1, sem2):
        wid = lax.axis_index("s") * 2 + lax.axis_index("c")
        base = wid * (B // NW)
        for c in range(nch):
            off = base + c * CH
            pltpu.sync_copy(idx_hbm.at[pl.ds(off, CH)], idx_v)
            ck = pltpu.async_copy(ktab_hbm.at[idx_v], krows, sem1)
            cv = pltpu.async_copy(vtab_hbm.at[idx_v], vrows, sem2)
            ck.wait()
            cv.wait()
            pltpu.sync_copy(krows, ko_hbm.at[pl.ds(off, CH)])
            pltpu.sync_copy(vrows, vo_hbm.at[pl.ds(off, CH)])

    return gk(ktab, vtab, idx)


def _stage_c(q_ref, k_ref, v_ref, base_ref, w2_ref, o_ref, *, P):
    q = q_ref[0]
    kk = k_ref[0].reshape(P, 32, 32)
    vv = v_ref[0].reshape(P, 32, 32)
    qb = jnp.broadcast_to(q[:, :, None, :], (P, 4, 32, 32))
    kb = jnp.broadcast_to(kk[:, None, :, :], (P, 4, 32, 32))
    s = jnp.sum(qb * kb, axis=-1) * SCALE
    mx = jnp.max(s, axis=-1, keepdims=True)
    e = jnp.exp(s - mx)
    den = jnp.sum(e, axis=-1, keepdims=True)
    pb = jnp.broadcast_to(e[..., None], (P, 4, 32, 32))
    vb = jnp.broadcast_to(vv[:, None, :, :], (P, 4, 32, 32))
    m0 = jnp.sum(pb * vb, axis=2) / den
    o_ref[0] = base_ref[0][:, None, :] + w2_ref[0, 0] * m0


def kernel(q0, q1, q2, k0, k1, k2, v0, v1, v2, weight):
    f32 = jnp.float32
    q2t, k2t, v2t = _tok(q2), _tok(k2), _tok(v2)
    q0c, k0c, v0c = _cells(q0), _cells(k0), _cells(v0)

    m2, i2 = _pallas_call(
        _stage_a,
        grid=(NH,),
        in_specs=[pl.BlockSpec((1, 256, HD), lambda h: (h, 0, 0))] * 3,
        out_specs=[pl.BlockSpec((1, 256, HD), lambda h: (h, 0, 0)),
                   pl.BlockSpec((1, 256, 16), lambda h: (h, 0, 0))],
        out_shape=[jax.ShapeDtypeStruct((NH, 256, HD), f32),
                   jax.ShapeDtypeStruct((NH, 256, 16), jnp.int32)],
    )(q2t, k2t, v2t)

    q1r = q1.reshape(NH, HD, 1024)
    k1r = k1.reshape(NH, HD, 1024)
    v1r = v1.reshape(NH, HD, 1024)
    i2tok = jnp.broadcast_to(
        i2.reshape(NH, 16, 1, 16, 1, 16), (NH, 16, 2, 16, 2, 16)
    ).reshape(NH, 1024, 16)
    m1, i1 = _pallas_call(
        _stage_b,
        grid=(NH,),
        in_specs=[pl.BlockSpec((1, HD, 1024), lambda h: (h, 0, 0))] * 3
        + [pl.BlockSpec((1, 1024, 16), lambda h: (h, 0, 0))],
        out_specs=[pl.BlockSpec((1, 1024, HD), lambda h: (h, 0, 0)),
                   pl.BlockSpec((1, 1024, 8), lambda h: (h, 0, 0))],
        out_shape=[jax.ShapeDtypeStruct((NH, 1024, HD), f32),
                   jax.ShapeDtypeStruct((NH, 1024, 8), jnp.int32)],
    )(q1r, k1r, v1r, i2tok)

    wsm = jax.nn.softmax(weight, axis=0)
    m2up = jnp.broadcast_to(
        m2.reshape(NH, 16, 1, 16, 1, HD), (NH, 16, 2, 16, 2, HD)
    ).reshape(NH, 1024, HD)
    base = wsm[0] * m2up + wsm[1] * m1
    i1tok = i1

    def cell_rows(x):
        t = x.reshape(NH, 4, 1024, HD).transpose(0, 2, 1, 3)
        t = jnp.concatenate([t, jnp.zeros((NH, 1024, 4, 8), jnp.float32)], axis=-1)
        return t.reshape(NH * 1024, 128)
    ktab = cell_rows(k0c)
    vtab = cell_rows(v0c)
    hoff = (jnp.arange(NH, dtype=jnp.int32) * 1024)[:, None, None]
    idx = (i1tok + hoff).reshape(NH * 1024 * 8)
    kg, vg = _sc_gather2(ktab, vtab, idx)
    kg = kg.reshape(NH, 1024, 8, 4, 32)
    vg = vg.reshape(NH, 1024, 8, 4, 32)

    P = 256
    npb = 1024 // P
    q0p = q0c.reshape(NH, 4, 1024, HD).transpose(0, 2, 1, 3)
    q0p = jnp.concatenate([q0p, jnp.zeros((NH, 1024, 4, 8), jnp.float32)], axis=-1)
    basep = jnp.concatenate([base, jnp.zeros((NH, 1024, 8), jnp.float32)], axis=-1)
    w2 = wsm[2].reshape(1, 1)
    outf = _pallas_call(
        functools.partial(_stage_c, P=P),
        grid=(NH, npb),
        in_specs=[
            pl.BlockSpec((1, P, 4, 32), lambda h, pb: (h, pb, 0, 0)),
            pl.BlockSpec((1, P, 8, 4, 32), lambda h, pb: (h, pb, 0, 0, 0)),
            pl.BlockSpec((1, P, 8, 4, 32), lambda h, pb: (h, pb, 0, 0, 0)),
            pl.BlockSpec((1, P, 32), lambda h, pb: (h, pb, 0)),
            pl.BlockSpec((1, 1), lambda h, pb: (0, 0)),
        ],
        out_specs=pl.BlockSpec((1, P, 4, 32), lambda h, pb: (h, pb, 0, 0)),
        out_shape=jax.ShapeDtypeStruct((NH, 1024, 4, 32), f32),
    )(q0p, kg, vg, basep, w2)

    out = outf[..., :HD].reshape(NH, 32, 32, 2, 2, HD).transpose(0, 1, 3, 2, 4, 5)
    out = out.reshape(NH, 4096, HD).transpose(1, 0, 2)[None]
    return out
